# baseline (speedup 1.0000x reference)
import functools

import jax
import jax.numpy as jnp
from jax import lax
from jax.experimental import pallas as pl
from jax.experimental.pallas import tpu as pltpu
from jax.experimental.pallas import tpu_sc as plsc

N_NODES = 10000
N_EDGES = 320000
D_IN = 128
D_OUT = 128

NC = 2
NS = 16
NW = NC * NS
CHUNK = 96
NCHUNKS = 107
E_PAD = NW * NCHUNKS * CHUNK
PK = 2 * CHUNK
N_PAD = 10240
ROWS_PER_TILE = N_PAD // NS
LANES = 16
NGROUPS = CHUNK // LANES


@functools.partial(
    pl.kernel,
    out_type=jax.ShapeDtypeStruct((NC, N_PAD, D_IN), jnp.float32),
    mesh=plsc.VectorSubcoreMesh(core_axis_name="c", subcore_axis_name="s"),
    scratch_types=[
        pltpu.VMEM((PK,), jnp.int32),
        pltpu.VMEM((PK,), jnp.int32),
        pltpu.VMEM((CHUNK,), jnp.float32),
        pltpu.VMEM((CHUNK,), jnp.float32),
        pltpu.VMEM((CHUNK,), jnp.int32),
        pltpu.VMEM((CHUNK,), jnp.int32),
        pltpu.VMEM((CHUNK,), jnp.int32),
        pltpu.VMEM((CHUNK, D_IN), jnp.float32),
        pltpu.VMEM((CHUNK, D_IN), jnp.float32),
        pltpu.VMEM((CHUNK, D_IN), jnp.float32),
        pltpu.VMEM_SHARED((N_PAD, D_IN), jnp.float32),
        pltpu.SemaphoreType.DMA,
        pltpu.SemaphoreType.DMA,
        pltpu.SemaphoreType.DMA,
        pltpu.SemaphoreType.DMA,
        pltpu.SemaphoreType.DMA,
        pltpu.SemaphoreType.DMA,
        pltpu.SemaphoreType.DMA,
        pltpu.SemaphoreType.DMA,
    ],
)
def _sc_spmm(h_hbm, packed_hbm, val_hbm, out_hbm,
             p0, p1, v0, v1, r0, r1, r2, b0, b1, b2, acc,
             g0, g1, g2, s0, s1, s2, i0, i1):
    cid = lax.axis_index("c")
    sid = lax.axis_index("s")
    wid = sid * NC + cid
    base0 = wid * NCHUNKS

    zv = jnp.zeros((LANES,), jnp.float32)
    for e in range(CHUNK):
        for j in range(D_IN // LANES):
            b0[e, pl.ds(j * LANES, LANES)] = zv

    def zero_body(i, carry):
        pltpu.sync_copy(
            b0, acc.at[pl.ds(sid * ROWS_PER_TILE + i * CHUNK, CHUNK)])
        return carry

    lax.fori_loop(0, ROWS_PER_TILE // CHUNK, zero_body, 0)
    plsc.subcore_barrier()

    def start_load_idx(c, pv, vv_ref, sem):
        pltpu.async_copy(packed_hbm.at[pl.ds((base0 + c) * PK, PK)], pv, sem)
        pltpu.async_copy(
            val_hbm.at[pl.ds(base0 * CHUNK + c * CHUNK, CHUNK)], vv_ref, sem)

    def wait_load_idx(c, pv, vv_ref, sem):
        pltpu.make_async_copy(
            packed_hbm.at[pl.ds((base0 + c) * PK, PK)], pv, sem).wait()
        pltpu.make_async_copy(
            val_hbm.at[pl.ds(base0 * CHUNK + c * CHUNK, CHUNK)], vv_ref,
            sem).wait()

    def copy_row_idx(pv, rv):
        for i in range(NGROUPS):
            rv[pl.ds(i * LANES, LANES)] = pv[pl.ds(CHUNK + i * LANES, LANES)]

    def start_gather(pv, bv, sem):
        pltpu.async_copy(h_hbm.at[pv.at[pl.ds(0, CHUNK)]], bv, sem)

    def wait_gather(pv, bv, sem):
        pltpu.make_async_copy(h_hbm.at[pv.at[pl.ds(0, CHUNK)]], bv, sem).wait()

    def scale(bv, vv_ref):
        def group_body(g, carry):
            v16 = vv_ref[pl.ds(g * LANES, LANES)]
            for l in range(LANES):
                vv = jnp.full((LANES,), v16[l], jnp.float32)
                e = g * LANES + l
                for j in range(D_IN // LANES):
                    sl = pl.ds(j * LANES, LANES)
                    bv[e, sl] = bv[e, sl] * vv
            return carry

        lax.fori_loop(0, NGROUPS, group_body, 0)

    def start_scatter(bv, rv, sem):
        pltpu.async_copy(bv, acc.at[rv], sem, add=True)

    def wait_scatter(bv, rv, sem):
        pltpu.make_async_copy(bv, acc.at[rv], sem).wait()

    P, V, I = [p0, p1], [v0, v1], [i0, i1]
    R, B, G, S = [r0, r1, r2], [b0, b1, b2], [g0, g1, g2], [s0, s1, s2]

    def full_step(c, m2, m3, prefetch=True, wait_prev_scatter=True):
        n2, n3 = (m2 + 1) % 2, (m3 + 1) % 3
        wait_load_idx(c + 1, P[n2], V[n2], I[n2])
        if wait_prev_scatter:
            wait_scatter(B[n3], R[n3], S[n3])
        copy_row_idx(P[n2], R[n3])
        start_gather(P[n2], B[n3], G[n3])
        wait_gather(P[m2], B[m3], G[m3])
        scale(B[m3], V[m2])
        if prefetch:
            start_load_idx(c + 2, P[m2], V[m2], I[m2])
        start_scatter(B[m3], R[m3], S[m3])

    start_load_idx(0, p0, v0, i0)
    wait_load_idx(0, p0, v0, i0)
    copy_row_idx(p0, r0)
    start_gather(p0, b0, g0)
    start_load_idx(1, p1, v1, i1)
    full_step(0, 0, 0, wait_prev_scatter=False)
    full_step(1, 1, 1, wait_prev_scatter=False)

    def six_body(k, carry):
        for d in range(6):
            full_step(6 * k + 2 + d, d % 2, (2 + d) % 3)
        return carry

    lax.fori_loop(0, (NCHUNKS - 5) // 6, six_body, 0)

    full_step(NCHUNKS - 3, 0, 2)
    full_step(NCHUNKS - 2, 1, 0, prefetch=False)
    wait_gather(P[0], B[1], G[1])
    scale(B[1], V[0])
    wait_scatter(B[2], R[2], S[2])
    start_scatter(B[1], R[1], S[1])
    wait_scatter(B[0], R[0], S[0])
    wait_scatter(B[1], R[1], S[1])

    plsc.subcore_barrier()
    pltpu.sync_copy(acc.at[pl.ds(sid * ROWS_PER_TILE, ROWS_PER_TILE)],
                    out_hbm.at[cid, pl.ds(sid * ROWS_PER_TILE, ROWS_PER_TILE)])


_BM = 1000


def _tc_body(p_ref, w_ref, b_ref, o_ref):
    s = p_ref[0] + p_ref[1]
    acc = jnp.dot(s, w_ref[...], preferred_element_type=jnp.float32)
    o_ref[...] = jnp.maximum(acc + b_ref[...], 0.0)


def _tc_combine(partials, W, bias2d):
    return pl.pallas_call(
        _tc_body,
        grid=(N_NODES // _BM,),
        in_specs=[
            pl.BlockSpec((NC, _BM, D_IN), lambda i: (0, i, 0)),
            pl.BlockSpec((D_IN, D_OUT), lambda i: (0, 0)),
            pl.BlockSpec((1, D_OUT), lambda i: (0, 0)),
        ],
        out_specs=pl.BlockSpec((_BM, D_OUT), lambda i: (i, 0)),
        out_shape=jax.ShapeDtypeStruct((N_NODES, D_OUT), jnp.float32),
    )(partials, W, bias2d)


def kernel(A_edge_index, A_values, H, W, bias):
    row = A_edge_index[0]
    col = A_edge_index[1]
    pad = E_PAD - N_EDGES
    zi = jnp.zeros((pad,), jnp.int32)
    col_p = jnp.concatenate([col, zi])
    row_p = jnp.concatenate([row, zi])
    val_p = jnp.concatenate([A_values, jnp.zeros((pad,), jnp.float32)])
    nchunks_total = E_PAD // CHUNK
    packed = jnp.concatenate(
        [col_p.reshape(nchunks_total, CHUNK),
         row_p.reshape(nchunks_total, CHUNK)], axis=1).reshape(-1)
    partials = _sc_spmm(H, packed, val_p)
    return _tc_combine(partials, W, bias.reshape(1, D_OUT))


# speedup vs baseline: 2.3519x; 2.3519x over previous
import functools

import jax
import jax.numpy as jnp
from jax import lax
from jax.experimental import pallas as pl
from jax.experimental.pallas import tpu as pltpu
from jax.experimental.pallas import tpu_sc as plsc

N_NODES = 10000
N_EDGES = 320000
D_IN = 128
D_OUT = 128

NC = 2
NS = 16
NW = NC * NS
CHUNK = 80
NCHUNKS = 125
E_PAD = NW * NCHUNKS * CHUNK
PK = 2 * CHUNK
N_PAD = 10240
ROWS_PER_TILE = N_PAD // NS
LANES = 16
NGROUPS = CHUNK // LANES


@functools.partial(
    pl.kernel,
    out_type=jax.ShapeDtypeStruct((NC, N_PAD, D_IN), jnp.float32),
    mesh=plsc.VectorSubcoreMesh(core_axis_name="c", subcore_axis_name="s"),
    scratch_types=[
        pltpu.VMEM((PK,), jnp.int32),
        pltpu.VMEM((PK,), jnp.int32),
        pltpu.VMEM((CHUNK,), jnp.float32),
        pltpu.VMEM((CHUNK,), jnp.float32),
        pltpu.VMEM((CHUNK,), jnp.int32),
        pltpu.VMEM((CHUNK,), jnp.int32),
        pltpu.VMEM((CHUNK,), jnp.int32),
        pltpu.VMEM((CHUNK, D_IN), jnp.float32),
        pltpu.VMEM((CHUNK, D_IN), jnp.float32),
        pltpu.VMEM((CHUNK, D_IN), jnp.float32),
        pltpu.VMEM_SHARED((N_PAD, D_IN), jnp.float32),
        pltpu.SemaphoreType.DMA,
        pltpu.SemaphoreType.DMA,
        pltpu.SemaphoreType.DMA,
        pltpu.SemaphoreType.DMA,
        pltpu.SemaphoreType.DMA,
        pltpu.SemaphoreType.DMA,
        pltpu.SemaphoreType.DMA,
        pltpu.SemaphoreType.DMA,
    ],
)
def _sc_spmm(h_hbm, packed_hbm, val_hbm, out_hbm,
             p0, p1, v0, v1, r0, r1, r2, b0, b1, b2, acc,
             g0, g1, g2, s0, s1, s2, i0, i1):
    cid = lax.axis_index("c")
    sid = lax.axis_index("s")
    wid = sid * NC + cid
    base0 = wid * NCHUNKS

    zv = jnp.zeros((LANES,), jnp.float32)
    for e in range(CHUNK):
        for j in range(D_IN // LANES):
            b0[e, pl.ds(j * LANES, LANES)] = zv

    def zero_body(i, carry):
        pltpu.sync_copy(
            b0, acc.at[pl.ds(sid * ROWS_PER_TILE + i * CHUNK, CHUNK)])
        return carry

    lax.fori_loop(0, ROWS_PER_TILE // CHUNK, zero_body, 0)
    plsc.subcore_barrier()

    def start_load_idx(c, pv, vv_ref, sem):
        pltpu.async_copy(packed_hbm.at[pl.ds((base0 + c) * PK, PK)], pv, sem)
        pltpu.async_copy(
            val_hbm.at[pl.ds(base0 * CHUNK + c * CHUNK, CHUNK)], vv_ref, sem)

    def wait_load_idx(c, pv, vv_ref, sem):
        pltpu.make_async_copy(
            packed_hbm.at[pl.ds((base0 + c) * PK, PK)], pv, sem).wait()
        pltpu.make_async_copy(
            val_hbm.at[pl.ds(base0 * CHUNK + c * CHUNK, CHUNK)], vv_ref,
            sem).wait()

    def copy_row_idx(pv, rv):
        for i in range(NGROUPS):
            rv[pl.ds(i * LANES, LANES)] = pv[pl.ds(CHUNK + i * LANES, LANES)]

    def start_gather(pv, bv, sem):
        pltpu.async_copy(h_hbm.at[pv.at[pl.ds(0, CHUNK)]], bv, sem)

    def wait_gather(pv, bv, sem):
        pltpu.make_async_copy(h_hbm.at[pv.at[pl.ds(0, CHUNK)]], bv, sem).wait()

    def scale(bv, vv_ref):
        def group_body(g, carry):
            v16 = vv_ref[pl.ds(g * LANES, LANES)]
            for l in range(LANES):
                vv = jnp.full((LANES,), v16[l], jnp.float32)
                e = g * LANES + l
                for j in range(D_IN // LANES):
                    sl = pl.ds(j * LANES, LANES)
                    bv[e, sl] = bv[e, sl] * vv
            return carry

        lax.fori_loop(0, NGROUPS, group_body, 0)

    def start_scatter(bv, rv, sem):
        pltpu.async_copy(bv, acc.at[rv], sem, add=True)

    def wait_scatter(bv, rv, sem):
        pltpu.make_async_copy(bv, acc.at[rv], sem).wait()

    P, V, I = [p0, p1], [v0, v1], [i0, i1]
    R, B, G, S = [r0, r1, r2], [b0, b1, b2], [g0, g1, g2], [s0, s1, s2]

    def full_step(c, m2, m3, prefetch=True, wait_prev_scatter=True):
        n2, n3 = (m2 + 1) % 2, (m3 + 1) % 3
        wait_load_idx(c + 1, P[n2], V[n2], I[n2])
        if wait_prev_scatter:
            wait_scatter(B[n3], R[n3], S[n3])
        copy_row_idx(P[n2], R[n3])
        start_gather(P[n2], B[n3], G[n3])
        wait_gather(P[m2], B[m3], G[m3])
        scale(B[m3], V[m2])
        if prefetch:
            start_load_idx(c + 2, P[m2], V[m2], I[m2])
        start_scatter(B[m3], R[m3], S[m3])

    start_load_idx(0, p0, v0, i0)
    wait_load_idx(0, p0, v0, i0)
    copy_row_idx(p0, r0)
    start_gather(p0, b0, g0)
    start_load_idx(1, p1, v1, i1)
    full_step(0, 0, 0, wait_prev_scatter=False)
    full_step(1, 1, 1, wait_prev_scatter=False)

    def six_body(k, carry):
        for d in range(6):
            full_step(6 * k + 2 + d, d % 2, (2 + d) % 3)
        return carry

    lax.fori_loop(0, (NCHUNKS - 5) // 6, six_body, 0)

    full_step(NCHUNKS - 3, 0, 2)
    full_step(NCHUNKS - 2, 1, 0, prefetch=False)
    wait_gather(P[0], B[1], G[1])
    scale(B[1], V[0])
    wait_scatter(B[2], R[2], S[2])
    start_scatter(B[1], R[1], S[1])
    wait_scatter(B[0], R[0], S[0])
    wait_scatter(B[1], R[1], S[1])

    plsc.subcore_barrier()
    pltpu.sync_copy(acc.at[pl.ds(sid * ROWS_PER_TILE, ROWS_PER_TILE)],
                    out_hbm.at[cid, pl.ds(sid * ROWS_PER_TILE, ROWS_PER_TILE)])


_BM = 1000


def _tc_body(p_ref, w_ref, b_ref, o_ref):
    s = p_ref[0] + p_ref[1]
    acc = jnp.dot(s, w_ref[...], preferred_element_type=jnp.float32)
    o_ref[...] = jnp.maximum(acc + b_ref[...], 0.0)


def _tc_combine(partials, W, bias2d):
    return pl.pallas_call(
        _tc_body,
        grid=(N_NODES // _BM,),
        in_specs=[
            pl.BlockSpec((NC, _BM, D_IN), lambda i: (0, i, 0)),
            pl.BlockSpec((D_IN, D_OUT), lambda i: (0, 0)),
            pl.BlockSpec((1, D_OUT), lambda i: (0, 0)),
        ],
        out_specs=pl.BlockSpec((_BM, D_OUT), lambda i: (i, 0)),
        out_shape=jax.ShapeDtypeStruct((N_NODES, D_OUT), jnp.float32),
    )(partials, W, bias2d)


def kernel(A_edge_index, A_values, H, W, bias):
    row = A_edge_index[0]
    col = A_edge_index[1]
    pad = E_PAD - N_EDGES
    zi = jnp.zeros((pad,), jnp.int32)
    col_p = jnp.concatenate([col, zi])
    row_p = jnp.concatenate([row, zi])
    val_p = jnp.concatenate([A_values, jnp.zeros((pad,), jnp.float32)])
    nchunks_total = E_PAD // CHUNK
    packed = jnp.concatenate(
        [col_p.reshape(nchunks_total, CHUNK),
         row_p.reshape(nchunks_total, CHUNK)], axis=1).reshape(-1)
    partials = _sc_spmm(H, packed, val_p)
    return _tc_combine(partials, W, bias.reshape(1, D_OUT))
